# baseline (speedup 1.0000x reference)
import jax
import jax.numpy as jnp
from jax.experimental import pallas as pl

N = 512
U = 64
B = 32
BC = 16
NTAP = 3
F32 = jnp.float32


def _split_w(W, xdim):
    F = xdim + U
    out = W.shape[1]
    W3 = W.reshape(F, NTAP, out)
    Wx = jnp.transpose(W3[:xdim], (1, 0, 2))
    Wh = jnp.transpose(W3[xdim:], (1, 0, 2))
    return Wx, Wh


def _kron_x0(Wx):
    eye = jnp.eye(BC, dtype=F32)
    return jax.vmap(lambda w: jnp.kron(eye, w))(Wx)


def _dcgru_body(inp_ref, adj_ref, hid_ref,
                kxg0_ref, whg0_ref, kxc0_ref, whc0_ref, bg0_ref, bc0_ref,
                wxg1_ref, whg1_ref, wxc1_ref, whc1_ref, bg1_ref, bc1_ref,
                wp_ref, bp_ref,
                out_ref, hs_ref):
    A = adj_ref[...]

    def mm(a, b):
        return jax.lax.dot_general(a.astype(jnp.bfloat16),
                                   b.astype(jnp.bfloat16),
                                   (((1,), (0,)), ((), ())),
                                   preferred_element_type=F32)

    def diffuse(z_nm):
        z1 = mm(A, z_nm)
        z2 = 2.0 * mm(A, z1) - z_nm
        return z_nm, z1, z2

    def rows(z_nm, c):
        return (z_nm.reshape(N, BC, c) + 0.0).reshape(N * BC, c)

    def nm(z_rows, c):
        return (z_rows.reshape(N, BC, c) + 0.0).reshape(N, BC * c)

    h0 = jnp.transpose(hid_ref[0].reshape(BC, N, U), (1, 0, 2))
    h0_rows = h0.reshape(N * BC, U)

    x_nm = jnp.transpose(inp_ref[...], (1, 0))
    x_taps = diffuse(x_nm)

    def gconv0(s_rows, kx, wh, bias, out_c):
        s0, s1, s2 = diffuse(nm(s_rows, U))
        acc = jnp.broadcast_to(bias[None, :], (N * BC, out_c))
        for m, s in enumerate((s0, s1, s2)):
            acc = acc + mm(rows(s, U), wh[m])
            acc = acc + rows(mm(x_taps[m], kx[m]), out_c)
        return acc

    g = jax.nn.sigmoid(gconv0(h0_rows, kxg0_ref[...], whg0_ref[...],
                              bg0_ref[...], 2 * U))
    r, u = g[:, :U], g[:, U:]
    c = jnp.tanh(gconv0(r * h0_rows, kxc0_ref[...], whc0_ref[...],
                        bc0_ref[...], U))
    h0n = u * h0_rows + (1.0 - u) * c

    h1 = jnp.transpose(hid_ref[1].reshape(BC, N, U), (1, 0, 2))
    h1_rows = h1.reshape(N * BC, U)
    x1_taps = diffuse(nm(h0n, U))

    def gconv1(s_rows, wx, wh, bias, out_c):
        s0, s1, s2 = diffuse(nm(s_rows, U))
        acc = jnp.broadcast_to(bias[None, :], (N * BC, out_c))
        for m, (s, xm) in enumerate(zip((s0, s1, s2), x1_taps)):
            acc = acc + mm(rows(s, U), wh[m]) + mm(rows(xm, U), wx[m])
        return acc

    g = jax.nn.sigmoid(gconv1(h1_rows, wxg1_ref[...], whg1_ref[...],
                              bg1_ref[...], 2 * U))
    r, u = g[:, :U], g[:, U:]
    c = jnp.tanh(gconv1(r * h1_rows, wxc1_ref[...], whc1_ref[...],
                        bc1_ref[...], U))
    h1n = u * h1_rows + (1.0 - u) * c

    hs_ref[0] = jnp.transpose(h0n.reshape(N, BC, U), (1, 0, 2)).reshape(BC, N * U)
    hs_ref[1] = jnp.transpose(h1n.reshape(N, BC, U), (1, 0, 2)).reshape(BC, N * U)
    proj = jnp.sum(h1n.reshape(N, BC, U) * wp_ref[...][None, :, :],
                   axis=-1) + bp_ref[0]
    out_ref[...] = jnp.transpose(proj, (1, 0))


def kernel(inputs, adj_mx, hidden_state, Wg0, bg0, Wc0, bc0,
           Wg1, bg1, Wc1, bc1, Wp, bp):
    wxg0, whg0 = _split_w(Wg0, 1)
    wxc0, whc0 = _split_w(Wc0, 1)
    kxg0 = _kron_x0(wxg0)
    kxc0 = _kron_x0(wxc0)
    wxg1, whg1 = _split_w(Wg1, U)
    wxc1, whc1 = _split_w(Wc1, U)

    full = lambda *shape: pl.BlockSpec(shape, lambda i: (0,) * len(shape))
    out, hs = pl.pallas_call(
        _dcgru_body,
        grid=(B // BC,),
        in_specs=[
            pl.BlockSpec((BC, N), lambda i: (i, 0)),
            full(N, N),
            pl.BlockSpec((2, BC, N * U), lambda i: (0, i, 0)),
            full(NTAP, BC, BC * 2 * U),
            full(NTAP, U, 2 * U),
            full(NTAP, BC, BC * U),
            full(NTAP, U, U),
            full(2 * U), full(U),
            full(NTAP, U, 2 * U),
            full(NTAP, U, 2 * U),
            full(NTAP, U, U),
            full(NTAP, U, U),
            full(2 * U), full(U),
            full(1, U), full(1),
        ],
        out_specs=(
            pl.BlockSpec((BC, N), lambda i: (i, 0)),
            pl.BlockSpec((2, BC, N * U), lambda i: (0, i, 0)),
        ),
        out_shape=(
            jax.ShapeDtypeStruct((B, N), F32),
            jax.ShapeDtypeStruct((2, B, N * U), F32),
        ),
    )(inputs, adj_mx, hidden_state,
      kxg0, whg0, kxc0, whc0, bg0, bc0,
      wxg1, whg1, wxc1, whc1, bg1, bc1,
      Wp.T, bp)
    return out, hs


# speedup vs baseline: 1.3858x; 1.3858x over previous
import jax
import jax.numpy as jnp
from jax.experimental import pallas as pl

N = 512
U = 64
B = 32
BC = 8
NTAP = 3
F32 = jnp.float32
BF16 = jnp.bfloat16


def _split_w(W, xdim):
    F = xdim + U
    out = W.shape[1]
    W3 = W.reshape(F, NTAP, out).astype(BF16)
    Wx = jnp.transpose(W3[:xdim], (1, 0, 2))
    Wh = jnp.transpose(W3[xdim:], (1, 0, 2))
    return Wx, Wh


def _dcgru_body(inp_ref, adj_ref, hid_ref,
                wxg0_ref, whg0_ref, wxc0_ref, whc0_ref, bg0_ref, bc0_ref,
                wxg1_ref, whg1_ref, wxc1_ref, whc1_ref, bg1_ref, bc1_ref,
                wp_ref, bp_ref,
                out_ref, hs_ref):
    A = adj_ref[...]
    Abf = A.astype(BF16)
    A2bf = (A * 2.0).astype(BF16)

    def mmf(a, b):
        return jax.lax.dot_general(a, b, (((1,), (0,)), ((), ())),
                                   preferred_element_type=F32)

    def mmb(a, b):
        return mmf(a, b).astype(BF16)

    def diffuse(z_nm):
        z1 = mmb(Abf, z_nm)
        z2 = mmb(A2bf, z1) - z_nm
        return z_nm, z1, z2

    def rows(z_nm, c):
        return (z_nm.reshape(N, BC, c) + jnp.zeros((), z_nm.dtype)
                ).reshape(N * BC, c)

    def nm(z_rows, c):
        return (z_rows.reshape(N, BC, c) + jnp.zeros((), z_rows.dtype)
                ).reshape(N, BC * c)

    h0 = jnp.transpose(hid_ref[0].reshape(BC, N, U), (1, 0, 2))
    h0_rows = h0.reshape(N * BC, U)

    x_nm = jnp.transpose(inp_ref[...], (1, 0)).astype(BF16)
    x_taps = diffuse(x_nm)
    x3 = [(t.astype(F32)).reshape(N, BC, 1) for t in x_taps]

    def gconv0(s_rows_bf, wx, wh, bias, out_c):
        s0, s1, s2 = diffuse(nm(s_rows_bf, U))
        acc = mmf(s_rows_bf, wh[0])
        acc = acc + mmf(rows(s1, U), wh[1])
        acc = acc + mmf(rows(s2, U), wh[2])
        acc3 = acc.reshape(N, BC, out_c) + bias[None, None, :]
        for m in range(NTAP):
            acc3 = acc3 + x3[m] * wx[m, 0].astype(F32)[None, None, :]
        return acc3.reshape(N * BC, out_c)

    h0_bf = h0_rows.astype(BF16)
    g = jax.nn.sigmoid(gconv0(h0_bf, wxg0_ref[...], whg0_ref[...],
                              bg0_ref[...], 2 * U))
    r, u = g[:, :U], g[:, U:]
    c = jnp.tanh(gconv0((r * h0_rows).astype(BF16), wxc0_ref[...],
                        whc0_ref[...], bc0_ref[...], U))
    h0n = u * h0_rows + (1.0 - u) * c

    h1 = jnp.transpose(hid_ref[1].reshape(BC, N, U), (1, 0, 2))
    h1_rows = h1.reshape(N * BC, U)
    h0n_bf = h0n.astype(BF16)
    xt0, xt1, xt2 = diffuse(nm(h0n_bf, U))
    x_rows = (h0n_bf, rows(xt1, U), rows(xt2, U))

    def gconv1(s_rows_bf, wx, wh, bias, out_c):
        s0, s1, s2 = diffuse(nm(s_rows_bf, U))
        acc = mmf(s_rows_bf, wh[0]) + mmf(x_rows[0], wx[0])
        acc = acc + mmf(rows(s1, U), wh[1]) + mmf(x_rows[1], wx[1])
        acc = acc + mmf(rows(s2, U), wh[2]) + mmf(x_rows[2], wx[2])
        return acc + bias[None, :]

    g = jax.nn.sigmoid(gconv1(h1_rows.astype(BF16), wxg1_ref[...],
                              whg1_ref[...], bg1_ref[...], 2 * U))
    r, u = g[:, :U], g[:, U:]
    c = jnp.tanh(gconv1((r * h1_rows).astype(BF16), wxc1_ref[...],
                        whc1_ref[...], bc1_ref[...], U))
    h1n = u * h1_rows + (1.0 - u) * c

    hs_ref[0] = jnp.transpose(h0n.reshape(N, BC, U), (1, 0, 2)).reshape(BC, N * U)
    hs_ref[1] = jnp.transpose(h1n.reshape(N, BC, U), (1, 0, 2)).reshape(BC, N * U)
    proj = jnp.sum(h1n.reshape(N, BC, U) * wp_ref[...][None, :, :],
                   axis=-1) + bp_ref[0]
    out_ref[...] = jnp.transpose(proj, (1, 0))


def kernel(inputs, adj_mx, hidden_state, Wg0, bg0, Wc0, bc0,
           Wg1, bg1, Wc1, bc1, Wp, bp):
    wxg0, whg0 = _split_w(Wg0, 1)
    wxc0, whc0 = _split_w(Wc0, 1)
    wxg1, whg1 = _split_w(Wg1, U)
    wxc1, whc1 = _split_w(Wc1, U)

    full = lambda *shape: pl.BlockSpec(shape, lambda i: (0,) * len(shape))
    out, hs = pl.pallas_call(
        _dcgru_body,
        grid=(B // BC,),
        in_specs=[
            pl.BlockSpec((BC, N), lambda i: (i, 0)),
            full(N, N),
            pl.BlockSpec((2, BC, N * U), lambda i: (0, i, 0)),
            full(NTAP, 1, 2 * U),
            full(NTAP, U, 2 * U),
            full(NTAP, 1, U),
            full(NTAP, U, U),
            full(2 * U), full(U),
            full(NTAP, U, 2 * U),
            full(NTAP, U, 2 * U),
            full(NTAP, U, U),
            full(NTAP, U, U),
            full(2 * U), full(U),
            full(1, U), full(1),
        ],
        out_specs=(
            pl.BlockSpec((BC, N), lambda i: (i, 0)),
            pl.BlockSpec((2, BC, N * U), lambda i: (0, i, 0)),
        ),
        out_shape=(
            jax.ShapeDtypeStruct((B, N), F32),
            jax.ShapeDtypeStruct((2, B, N * U), F32),
        ),
    )(inputs, adj_mx, hidden_state,
      wxg0, whg0, wxc0, whc0, bg0, bc0,
      wxg1, whg1, wxc1, whc1, bg1, bc1,
      Wp.T, bp)
    return out, hs


# speedup vs baseline: 1.8858x; 1.3608x over previous
import jax
import jax.numpy as jnp
from jax.experimental import pallas as pl

N = 512
U = 64
B = 32
BC = 8
NTAP = 3
F32 = jnp.float32
BF16 = jnp.bfloat16


def _split_w(W, xdim):
    F = xdim + U
    out = W.shape[1]
    W3 = W.reshape(F, NTAP, out).astype(BF16)
    Wx = jnp.transpose(W3[:xdim], (1, 0, 2))
    Wh = jnp.transpose(W3[xdim:], (1, 0, 2))
    return Wx, Wh


def _dcgru_body(inp_ref, adj_ref, hid_ref,
                wxg0_ref, wg0_ref, wxc0_ref, wc0_ref, bg0_ref, bc0_ref,
                wg1_ref, wc1_ref, bg1_ref, bc1_ref,
                wp_ref, bp_ref,
                out_ref, hs_ref):
    A = adj_ref[...]
    Abf = A.astype(BF16)
    A2bf = (A * 2.0).astype(BF16)

    def mmf(a, b):
        return jax.lax.dot_general(a, b, (((1,), (0,)), ((), ())),
                                   preferred_element_type=F32)

    def mmb(a, b):
        return mmf(a, b).astype(BF16)

    def diffuse(z_nm):
        z1 = mmb(Abf, z_nm)
        z2 = mmb(A2bf, z1) - z_nm
        return z_nm, z1, z2

    def nm(z_rows, c):
        return (z_rows.reshape(N, BC, c) + jnp.zeros((), z_rows.dtype)
                ).reshape(N, BC * c)

    def split3(z_nm):
        return z_nm.reshape(N, BC, U)

    def cat_rows(p3):
        return jnp.concatenate(p3, axis=2).reshape(N * BC, len(p3) * U)

    h0 = jnp.transpose(hid_ref[0].reshape(BC, N, U), (1, 0, 2))
    h0_rows = h0.reshape(N * BC, U)

    x_nm = jnp.transpose(inp_ref[...], (1, 0)).astype(BF16)
    x_taps = diffuse(x_nm)
    x3 = [(t.astype(F32)).reshape(N, BC, 1) for t in x_taps]

    def gconv0(s_rows_bf, wx, whcat, bias, out_c):
        s0, s1, s2 = diffuse(nm(s_rows_bf, U))
        acc = mmf(cat_rows([s_rows_bf.reshape(N, BC, U),
                            split3(s1), split3(s2)]), whcat)
        acc3 = acc.reshape(N, BC, out_c) + bias[None, None, :]
        for m in range(NTAP):
            acc3 = acc3 + x3[m] * wx[m, 0].astype(F32)[None, None, :]
        return acc3.reshape(N * BC, out_c)

    h0_bf = h0_rows.astype(BF16)
    g = jax.nn.sigmoid(gconv0(h0_bf, wxg0_ref[...], wg0_ref[...],
                              bg0_ref[...], 2 * U))
    r, u = g[:, :U], g[:, U:]
    c = jnp.tanh(gconv0((r * h0_rows).astype(BF16), wxc0_ref[...],
                        wc0_ref[...], bc0_ref[...], U))
    h0n = u * h0_rows + (1.0 - u) * c

    h1 = jnp.transpose(hid_ref[1].reshape(BC, N, U), (1, 0, 2))
    h1_rows = h1.reshape(N * BC, U)
    h0n_bf = h0n.astype(BF16)
    xt0, xt1, xt2 = diffuse(nm(h0n_bf, U))
    x3d = (h0n_bf.reshape(N, BC, U), split3(xt1), split3(xt2))

    def gconv1(s_rows_bf, wcat, bias, out_c):
        s0, s1, s2 = diffuse(nm(s_rows_bf, U))
        acc = mmf(cat_rows([s_rows_bf.reshape(N, BC, U),
                            split3(s1), split3(s2), *x3d]), wcat)
        return acc + bias[None, :]

    g = jax.nn.sigmoid(gconv1(h1_rows.astype(BF16),
                              wg1_ref[...], bg1_ref[...], 2 * U))
    r, u = g[:, :U], g[:, U:]
    c = jnp.tanh(gconv1((r * h1_rows).astype(BF16),
                        wc1_ref[...], bc1_ref[...], U))
    h1n = u * h1_rows + (1.0 - u) * c

    hs_ref[0] = jnp.transpose(h0n.reshape(N, BC, U), (1, 0, 2)).reshape(BC, N * U)
    hs_ref[1] = jnp.transpose(h1n.reshape(N, BC, U), (1, 0, 2)).reshape(BC, N * U)
    proj = jnp.sum(h1n.reshape(N, BC, U) * wp_ref[...][None, :, :],
                   axis=-1) + bp_ref[0]
    out_ref[...] = jnp.transpose(proj, (1, 0))


def kernel(inputs, adj_mx, hidden_state, Wg0, bg0, Wc0, bc0,
           Wg1, bg1, Wc1, bc1, Wp, bp):
    wxg0, whg0 = _split_w(Wg0, 1)
    wxc0, whc0 = _split_w(Wc0, 1)
    wxg1, whg1 = _split_w(Wg1, U)
    wxc1, whc1 = _split_w(Wc1, U)
    wg0 = jnp.concatenate(list(whg0), axis=0)
    wc0 = jnp.concatenate(list(whc0), axis=0)
    wg1 = jnp.concatenate(list(whg1) + list(wxg1), axis=0)
    wc1 = jnp.concatenate(list(whc1) + list(wxc1), axis=0)

    full = lambda *shape: pl.BlockSpec(shape, lambda i: (0,) * len(shape))
    out, hs = pl.pallas_call(
        _dcgru_body,
        grid=(B // BC,),
        in_specs=[
            pl.BlockSpec((BC, N), lambda i: (i, 0)),
            full(N, N),
            pl.BlockSpec((2, BC, N * U), lambda i: (0, i, 0)),
            full(NTAP, 1, 2 * U),
            full(NTAP * U, 2 * U),
            full(NTAP, 1, U),
            full(NTAP * U, U),
            full(2 * U), full(U),
            full(2 * NTAP * U, 2 * U),
            full(2 * NTAP * U, U),
            full(2 * U), full(U),
            full(1, U), full(1),
        ],
        out_specs=(
            pl.BlockSpec((BC, N), lambda i: (i, 0)),
            pl.BlockSpec((2, BC, N * U), lambda i: (0, i, 0)),
        ),
        out_shape=(
            jax.ShapeDtypeStruct((B, N), F32),
            jax.ShapeDtypeStruct((2, B, N * U), F32),
        ),
    )(inputs, adj_mx, hidden_state,
      wxg0, wg0, wxc0, wc0, bg0, bc0,
      wg1, wc1, bg1, bc1,
      Wp.T, bp)
    return out, hs


# speedup vs baseline: 2.1700x; 1.1507x over previous
import jax
import jax.numpy as jnp
from jax.experimental import pallas as pl

N = 512
U = 64
B = 32
BC = 16
NTAP = 3
F32 = jnp.float32
BF16 = jnp.bfloat16


def _split_w(W, xdim):
    F = xdim + U
    out = W.shape[1]
    W3 = W.reshape(F, NTAP, out).astype(BF16)
    Wx = jnp.transpose(W3[:xdim], (1, 0, 2))
    Wh = jnp.transpose(W3[xdim:], (1, 0, 2))
    return Wx, Wh


def _dcgru_body(inp_ref, adj_ref, hid_ref,
                wxg0_ref, wg0_ref, wxc0_ref, wc0_ref, bg0_ref, bc0_ref,
                wg1_ref, wc1_ref, bg1_ref, bc1_ref,
                wp_ref, bp_ref,
                out_ref, hs_ref):
    A = adj_ref[...]
    Abf = A.astype(BF16)
    A2bf = (A * 2.0).astype(BF16)

    def mmf(a, b):
        return jax.lax.dot_general(a, b, (((1,), (0,)), ((), ())),
                                   preferred_element_type=F32)

    def mmb(a, b):
        return mmf(a, b).astype(BF16)

    def diffuse(z_nm):
        z1 = mmb(Abf, z_nm)
        z2 = mmb(A2bf, z1) - z_nm
        return z_nm, z1, z2

    def nm(z_rows, c):
        return (z_rows.reshape(N, BC, c) + jnp.zeros((), z_rows.dtype)
                ).reshape(N, BC * c)

    def split3(z_nm):
        return z_nm.reshape(N, BC, U)

    def cat_rows(p3):
        return jnp.concatenate(p3, axis=2).reshape(N * BC, len(p3) * U)

    h0 = jnp.transpose(hid_ref[0].reshape(BC, N, U), (1, 0, 2))
    h0_rows = h0.reshape(N * BC, U)

    x_nm = jnp.transpose(inp_ref[...], (1, 0)).astype(BF16)
    x_taps = diffuse(x_nm)
    x3 = [(t.astype(F32)).reshape(N, BC, 1) for t in x_taps]

    def gconv0(s_rows_bf, wx, whcat, bias, out_c):
        s0, s1, s2 = diffuse(nm(s_rows_bf, U))
        acc = mmf(cat_rows([s_rows_bf.reshape(N, BC, U),
                            split3(s1), split3(s2)]), whcat)
        acc3 = acc.reshape(N, BC, out_c) + bias[None, None, :]
        for m in range(NTAP):
            acc3 = acc3 + x3[m] * wx[m, 0].astype(F32)[None, None, :]
        return acc3.reshape(N * BC, out_c)

    h0_bf = h0_rows.astype(BF16)
    g = jax.nn.sigmoid(gconv0(h0_bf, wxg0_ref[...], wg0_ref[...],
                              bg0_ref[...], 2 * U))
    r, u = g[:, :U], g[:, U:]
    c = jnp.tanh(gconv0((r * h0_rows).astype(BF16), wxc0_ref[...],
                        wc0_ref[...], bc0_ref[...], U))
    h0n = u * h0_rows + (1.0 - u) * c

    h1 = jnp.transpose(hid_ref[1].reshape(BC, N, U), (1, 0, 2))
    h1_rows = h1.reshape(N * BC, U)
    h0n_bf = h0n.astype(BF16)
    xt0, xt1, xt2 = diffuse(nm(h0n_bf, U))
    x3d = (h0n_bf.reshape(N, BC, U), split3(xt1), split3(xt2))

    def gconv1(s_rows_bf, wcat, bias, out_c):
        s0, s1, s2 = diffuse(nm(s_rows_bf, U))
        acc = mmf(cat_rows([s_rows_bf.reshape(N, BC, U),
                            split3(s1), split3(s2), *x3d]), wcat)
        return acc + bias[None, :]

    g = jax.nn.sigmoid(gconv1(h1_rows.astype(BF16),
                              wg1_ref[...], bg1_ref[...], 2 * U))
    r, u = g[:, :U], g[:, U:]
    c = jnp.tanh(gconv1((r * h1_rows).astype(BF16),
                        wc1_ref[...], bc1_ref[...], U))
    h1n = u * h1_rows + (1.0 - u) * c

    hs_ref[0] = jnp.transpose(h0n.reshape(N, BC, U), (1, 0, 2)).reshape(BC, N * U)
    hs_ref[1] = jnp.transpose(h1n.reshape(N, BC, U), (1, 0, 2)).reshape(BC, N * U)
    proj = jnp.sum(h1n.reshape(N, BC, U) * wp_ref[...][None, :, :],
                   axis=-1) + bp_ref[0]
    out_ref[...] = jnp.transpose(proj, (1, 0))


def kernel(inputs, adj_mx, hidden_state, Wg0, bg0, Wc0, bc0,
           Wg1, bg1, Wc1, bc1, Wp, bp):
    wxg0, whg0 = _split_w(Wg0, 1)
    wxc0, whc0 = _split_w(Wc0, 1)
    wxg1, whg1 = _split_w(Wg1, U)
    wxc1, whc1 = _split_w(Wc1, U)
    wg0 = jnp.concatenate(list(whg0), axis=0)
    wc0 = jnp.concatenate(list(whc0), axis=0)
    wg1 = jnp.concatenate(list(whg1) + list(wxg1), axis=0)
    wc1 = jnp.concatenate(list(whc1) + list(wxc1), axis=0)

    full = lambda *shape: pl.BlockSpec(shape, lambda i: (0,) * len(shape))
    out, hs = pl.pallas_call(
        _dcgru_body,
        grid=(B // BC,),
        in_specs=[
            pl.BlockSpec((BC, N), lambda i: (i, 0)),
            full(N, N),
            pl.BlockSpec((2, BC, N * U), lambda i: (0, i, 0)),
            full(NTAP, 1, 2 * U),
            full(NTAP * U, 2 * U),
            full(NTAP, 1, U),
            full(NTAP * U, U),
            full(2 * U), full(U),
            full(2 * NTAP * U, 2 * U),
            full(2 * NTAP * U, U),
            full(2 * U), full(U),
            full(1, U), full(1),
        ],
        out_specs=(
            pl.BlockSpec((BC, N), lambda i: (i, 0)),
            pl.BlockSpec((2, BC, N * U), lambda i: (0, i, 0)),
        ),
        out_shape=(
            jax.ShapeDtypeStruct((B, N), F32),
            jax.ShapeDtypeStruct((2, B, N * U), F32),
        ),
    )(inputs, adj_mx, hidden_state,
      wxg0, wg0, wxc0, wc0, bg0, bc0,
      wg1, wc1, bg1, bc1,
      Wp.T, bp)
    return out, hs
